# Initial kernel scaffold; baseline (speedup 1.0000x reference)
import jax
import jax.numpy as jnp
from jax.experimental import pallas as pl


def kernel(h, x, edge_index, edge_attr, W1, b1, W2, b2, W3, b3, W4, W5, b5, W6, b6):
    raise NotImplementedError("write your pallas kernel here")


# speedup vs baseline: 3.3652x; 3.3652x over previous
import functools

import jax
import jax.numpy as jnp
from jax import lax
from jax.experimental import pallas as pl
from jax.experimental.pallas import tpu as pltpu
from jax.experimental.pallas import tpu_sc as plsc

N = 10000
E = 320000
D = 128
ED = 16
H = 64
XP = 16
G = 128
NGROUPS = E // G
NC = 2
NS = 16
NW = NC * NS
RPT = N // NS

_sc_mesh = plsc.VectorSubcoreMesh(core_axis_name="c", subcore_axis_name="s",
                                  num_cores=NC)
_sc_params = pltpu.CompilerParams(use_tc_tiling_on_sc=False)


def _pre_body(h_ref, w1r_ref, w1c_ref, w5h_ref, hr_ref, hc_ref, p5_ref):
    h = h_ref[...]
    hr_ref[...] = jnp.dot(h, w1r_ref[...], preferred_element_type=jnp.float32)
    hc_ref[...] = jnp.dot(h, w1c_ref[...], preferred_element_type=jnp.float32)
    p5_ref[...] = jnp.dot(h, w5h_ref[...], preferred_element_type=jnp.float32)


def _pre_call(h, w1r, w1c, w5h):
    BN = 2000
    grid = (N // BN,)
    return pl.pallas_call(
        _pre_body,
        grid=grid,
        in_specs=[
            pl.BlockSpec((BN, D), lambda i: (i, 0)),
            pl.BlockSpec((D, H), lambda i: (0, 0)),
            pl.BlockSpec((D, H), lambda i: (0, 0)),
            pl.BlockSpec((D, H), lambda i: (0, 0)),
        ],
        out_specs=[
            pl.BlockSpec((BN, H), lambda i: (i, 0)),
            pl.BlockSpec((BN, H), lambda i: (i, 0)),
            pl.BlockSpec((BN, H), lambda i: (i, 0)),
        ],
        out_shape=[
            jax.ShapeDtypeStruct((N, H), jnp.float32),
            jax.ShapeDtypeStruct((N, H), jnp.float32),
            jax.ShapeDtypeStruct((N, H), jnp.float32),
        ],
    )(h, w1r, w1c, w5h)


@functools.partial(
    pl.kernel,
    out_type=(jax.ShapeDtypeStruct((E, H), jnp.float32),
              jax.ShapeDtypeStruct((E, XP), jnp.float32)),
    mesh=_sc_mesh,
    scratch_types=[
        pltpu.VMEM((1, G), jnp.int32),
        pltpu.VMEM((1, G), jnp.int32),
        pltpu.VMEM((G, H), jnp.float32),
        pltpu.VMEM((G, H), jnp.float32),
        pltpu.VMEM((G, XP), jnp.float32),
        pltpu.VMEM((G, XP), jnp.float32),
        pltpu.SemaphoreType.DMA,
        pltpu.SemaphoreType.DMA,
        pltpu.SemaphoreType.DMA,
        pltpu.SemaphoreType.DMA,
    ],
    compiler_params=_sc_params,
)
def _gather_kernel(row2d, col2d, hr_hbm, hc_hbm, xp_hbm, g_hbm, diff_hbm,
                   ridx, cidx, bufr, bufc, bufxr, bufxc,
                   sem1, sem2, sem3, sem4):
    c = lax.axis_index("c")
    s = lax.axis_index("s")
    w = s * NC + c
    n_w = (NGROUPS - w + NW - 1) // NW

    def body(i, _):
        g = w + i * NW
        base = g * G
        pltpu.sync_copy(row2d.at[pl.ds(g, 1)], ridx)
        pltpu.sync_copy(col2d.at[pl.ds(g, 1)], cidx)
        cp1 = pltpu.async_copy(hr_hbm.at[ridx.at[0]], bufr, sem1)
        cp2 = pltpu.async_copy(hc_hbm.at[cidx.at[0]], bufc, sem2)
        cp3 = pltpu.async_copy(xp_hbm.at[ridx.at[0]], bufxr, sem3)
        cp4 = pltpu.async_copy(xp_hbm.at[cidx.at[0]], bufxc, sem4)
        cp1.wait()
        cp2.wait()
        cp3.wait()
        cp4.wait()

        def addrow(r, carry):
            for k in range(H // 16):
                bufr[r, pl.ds(k * 16, 16)] = (bufr[r, pl.ds(k * 16, 16)]
                                              + bufc[r, pl.ds(k * 16, 16)])
            bufxr[r, pl.ds(0, 16)] = (bufxr[r, pl.ds(0, 16)]
                                      - bufxc[r, pl.ds(0, 16)])
            return carry

        lax.fori_loop(0, G, addrow, 0)
        pltpu.sync_copy(bufr, g_hbm.at[pl.ds(base, G)])
        pltpu.sync_copy(bufxr, diff_hbm.at[pl.ds(base, G)])
        return _

    lax.fori_loop(0, n_w, body, 0)


def _edge_body(g_ref, diff_ref, attr_ref, w1a_ref, wrad_ref, b1_ref,
               w2_ref, b2_ref, w3_ref, b3_ref, w4_ref, m_ref, trans_ref):
    diff = diff_ref[...]
    radial = jnp.sum(diff * diff, axis=1, keepdims=True)
    t1 = (g_ref[...] + radial * wrad_ref[...]
          + jnp.dot(attr_ref[...], w1a_ref[...],
                    preferred_element_type=jnp.float32)
          + b1_ref[...])
    a = t1 * jax.nn.sigmoid(t1)
    m = jnp.dot(a, w2_ref[...], preferred_element_type=jnp.float32) + b2_ref[...]
    u0 = jnp.dot(m, w3_ref[...], preferred_element_type=jnp.float32) + b3_ref[...]
    u = u0 * jax.nn.sigmoid(u0)
    phi = jnp.sum(u * w4_ref[...], axis=1, keepdims=True)
    m_ref[...] = m
    trans_ref[...] = diff * phi


def _edge_call(g, diff, attr, w1a, wrad, b1, w2, b2, w3, b3, w4row):
    BE = 4000
    grid = (E // BE,)
    full = lambda i: (0, 0)
    return pl.pallas_call(
        _edge_body,
        grid=grid,
        in_specs=[
            pl.BlockSpec((BE, H), lambda i: (i, 0)),
            pl.BlockSpec((BE, XP), lambda i: (i, 0)),
            pl.BlockSpec((BE, ED), lambda i: (i, 0)),
            pl.BlockSpec((ED, H), full),
            pl.BlockSpec((1, H), full),
            pl.BlockSpec((1, H), full),
            pl.BlockSpec((H, H), full),
            pl.BlockSpec((1, H), full),
            pl.BlockSpec((H, H), full),
            pl.BlockSpec((1, H), full),
            pl.BlockSpec((1, H), full),
        ],
        out_specs=[
            pl.BlockSpec((BE, H), lambda i: (i, 0)),
            pl.BlockSpec((BE, XP), lambda i: (i, 0)),
        ],
        out_shape=[
            jax.ShapeDtypeStruct((E, H), jnp.float32),
            jax.ShapeDtypeStruct((E, XP), jnp.float32),
        ],
    )(g, diff, attr, w1a, wrad, b1, w2, b2, w3, b3, w4row)


@functools.partial(
    pl.kernel,
    out_type=(jax.ShapeDtypeStruct((NC, N, H), jnp.float32),
              jax.ShapeDtypeStruct((NC, N, XP), jnp.float32)),
    mesh=_sc_mesh,
    scratch_types=[
        pltpu.VMEM((1, G), jnp.int32),
        pltpu.VMEM((G, H), jnp.float32),
        pltpu.VMEM((G, XP), jnp.float32),
        pltpu.VMEM_SHARED((N, H), jnp.float32),
        pltpu.VMEM_SHARED((N, XP), jnp.float32),
        pltpu.SemaphoreType.DMA,
        pltpu.SemaphoreType.DMA,
    ],
    compiler_params=_sc_params,
)
def _scatter_kernel(row2d, m_hbm, trans_hbm, zh_hbm, zx_hbm,
                    aggh_out, aggx_out,
                    ridx, bufm, bufx, sh_h, sh_x, sem1, sem2):
    c = lax.axis_index("c")
    s = lax.axis_index("s")
    w = s * NC + c
    n_w = (NGROUPS - w + NW - 1) // NW

    pltpu.sync_copy(zh_hbm, sh_h.at[pl.ds(s * RPT, RPT)])
    pltpu.sync_copy(zx_hbm, sh_x.at[pl.ds(s * RPT, RPT)])
    plsc.subcore_barrier()

    def body(i, _):
        g = w + i * NW
        base = g * G
        pltpu.sync_copy(row2d.at[pl.ds(g, 1)], ridx)
        cp1 = pltpu.async_copy(m_hbm.at[pl.ds(base, G)], bufm, sem1)
        cp2 = pltpu.async_copy(trans_hbm.at[pl.ds(base, G)], bufx, sem2)
        cp1.wait()
        cp2.wait()
        pltpu.sync_copy(bufm, sh_h.at[ridx.at[0]], add=True)
        pltpu.sync_copy(bufx, sh_x.at[ridx.at[0]], add=True)
        return _

    lax.fori_loop(0, n_w, body, 0)
    plsc.subcore_barrier()
    pltpu.sync_copy(sh_h.at[pl.ds(s * RPT, RPT)],
                    aggh_out.at[c].at[pl.ds(s * RPT, RPT)])
    pltpu.sync_copy(sh_x.at[pl.ds(s * RPT, RPT)],
                    aggx_out.at[c].at[pl.ds(s * RPT, RPT)])


def _node_body(h_ref, p5_ref, aggh_ref, aggx_ref, xp_ref,
               w5b_ref, b5_ref, w6_ref, b6_ref, hout_ref, xacc_ref):
    aggh = aggh_ref[0] + aggh_ref[1]
    z0 = (p5_ref[...]
          + jnp.dot(aggh, w5b_ref[...], preferred_element_type=jnp.float32)
          + b5_ref[...])
    z = z0 * jax.nn.sigmoid(z0)
    hout_ref[...] = (h_ref[...]
                     + jnp.dot(z, w6_ref[...],
                               preferred_element_type=jnp.float32)
                     + b6_ref[...])
    xacc_ref[...] = xp_ref[...] + aggx_ref[0] + aggx_ref[1]


def _node_call(h, p5, aggh_p, aggx_p, xpad, w5b, b5, w6, b6):
    BN = 2000
    grid = (N // BN,)
    full = lambda i: (0, 0)
    return pl.pallas_call(
        _node_body,
        grid=grid,
        in_specs=[
            pl.BlockSpec((BN, D), lambda i: (i, 0)),
            pl.BlockSpec((BN, H), lambda i: (i, 0)),
            pl.BlockSpec((NC, BN, H), lambda i: (0, i, 0)),
            pl.BlockSpec((NC, BN, XP), lambda i: (0, i, 0)),
            pl.BlockSpec((BN, XP), lambda i: (i, 0)),
            pl.BlockSpec((H, H), full),
            pl.BlockSpec((1, H), full),
            pl.BlockSpec((H, D), full),
            pl.BlockSpec((1, D), full),
        ],
        out_specs=[
            pl.BlockSpec((BN, D), lambda i: (i, 0)),
            pl.BlockSpec((BN, XP), lambda i: (i, 0)),
        ],
        out_shape=[
            jax.ShapeDtypeStruct((N, D), jnp.float32),
            jax.ShapeDtypeStruct((N, XP), jnp.float32),
        ],
    )(h, p5, aggh_p, aggx_p, xpad, w5b, b5, w6, b6)


def kernel(h, x, edge_index, edge_attr, W1, b1, W2, b2, W3, b3, W4, W5, b5,
           W6, b6):
    row2d = edge_index[0].reshape(NGROUPS, G)
    col2d = edge_index[1].reshape(NGROUPS, G)
    xpad = jnp.pad(x, ((0, 0), (0, XP - 3)))

    w1r = W1[:D]
    w1c = W1[D:2 * D]
    wrad = W1[2 * D:2 * D + 1]
    w1a = W1[2 * D + 1:]
    w4row = W4.reshape(1, H)

    hr, hc, p5 = _pre_call(h, w1r, w1c, W5[:D])
    g, diff = _gather_kernel(row2d, col2d, hr, hc, xpad)
    m, trans = _edge_call(g, diff, edge_attr, w1a, wrad, b1.reshape(1, H),
                          W2, b2.reshape(1, H), W3, b3.reshape(1, H), w4row)
    zh = jnp.zeros((RPT, H), jnp.float32)
    zx = jnp.zeros((RPT, XP), jnp.float32)
    aggh_p, aggx_p = _scatter_kernel(row2d, m, trans, zh, zx)
    h_out, xacc = _node_call(h, p5, aggh_p, aggx_p, xpad, W5[D:],
                             b5.reshape(1, H), W6, b6.reshape(1, D))
    x_out = xacc[:, :3]
    return (h_out, x_out)
